# Initial kernel scaffold; baseline (speedup 1.0000x reference)
import jax
import jax.numpy as jnp
from jax.experimental import pallas as pl


def kernel(cc_matrix, cc_edges, dd_matrix, dd_edges, x_cir, x_dis, W1c, b1c, Wgc, asrc_c, adst_c, We_c, ae_c, bg_c, W2c, b2c, W1d, b1d, Wgd, asrc_d, adst_d, We_d, ae_d, bg_d, W2d, b2d, Wcnn_c, bcnn_c, Wcnn_d, bcnn_d):
    raise NotImplementedError("write your pallas kernel here")


# speedup vs baseline: 60.2355x; 60.2355x over previous
import functools
import jax
import jax.numpy as jnp
from jax import lax
from jax.experimental import pallas as pl

N_CIR = 585
N_DIS = 88
M_CIR = 640
M_DIS = 128
H = 8
C = 128


def _hist_body(nb, chunk, m, edges_ref, out_ref):
    out_ref[...] = jnp.zeros_like(out_ref)

    def step(i, _):
        c0 = i * chunk
        rows = edges_ref[0:1, pl.ds(c0, chunk)]
        cols = edges_ref[1:2, pl.ds(c0, chunk)]
        iota = lax.broadcasted_iota(jnp.int32, (m, chunk), 0)
        rt = jnp.where(iota == jnp.broadcast_to(rows, (m, chunk)),
                       1.0, 0.0).astype(jnp.bfloat16)
        ct = jnp.where(iota == jnp.broadcast_to(cols, (m, chunk)),
                       1.0, 0.0).astype(jnp.bfloat16)
        out_ref[...] += lax.dot_general(
            ct, rt, (((1,), (1,)), ((), ())),
            preferred_element_type=jnp.float32)
        return 0

    lax.fori_loop(0, nb, step, 0)


def _hist(edges, m, chunk):
    nb = edges.shape[1] // chunk
    return pl.pallas_call(
        functools.partial(_hist_body, nb, chunk, m),
        out_shape=jax.ShapeDtypeStruct((m, m), jnp.float32),
    )(edges)


def _gcn(st, xw, b):
    deg = jnp.sum(st, axis=1, keepdims=True) + 1.0
    dinv = lax.rsqrt(deg)
    xs = xw * dinv
    out = lax.dot_general(st, xs, (((1,), (0,)), ((), ())),
                          preferred_element_type=jnp.float32) * dinv
    return jax.nn.relu(out + dinv * dinv * xw + b)


def _gat(cnt_t, mat_t, x1, wgt, msrc, mdst, msrct, coeff, einv, bg):
    m = x1.shape[0]
    xs = lax.dot_general(x1, wgt, (((1,), (0,)), ((), ())),
                         preferred_element_type=jnp.float32)
    a_src = lax.dot_general(x1, msrc, (((1,), (0,)), ((), ())),
                            preferred_element_type=jnp.float32)
    a_dst = lax.dot_general(x1, mdst, (((1,), (0,)), ((), ())),
                            preferred_element_type=jnp.float32)
    a_src_t = lax.dot_general(msrct, x1, (((1,), (1,)), ((), ())),
                              preferred_element_type=jnp.float32)
    st = cnt_t * mat_t
    mean_ea = jnp.sum(st, axis=1, keepdims=True).sum(axis=0, keepdims=True) * einv
    present = cnt_t > 0.0
    acc = jnp.zeros((m, C), jnp.float32)
    for h in range(H):
        ch = coeff[0:1, h:h + 1]
        alpha = a_dst[:, h:h + 1] + a_src_t[h:h + 1, :] + mat_t * ch
        alpha = jnp.where(alpha > 0, alpha, 0.2 * alpha)
        aloop = a_src[:, h:h + 1] + a_dst[:, h:h + 1] + mean_ea * ch
        aloop = jnp.where(aloop > 0, aloop, 0.2 * aloop)
        amax = jnp.max(jnp.where(present, alpha, -1e30), axis=1, keepdims=True)
        amax = jnp.maximum(amax, aloop)
        ex = cnt_t * jnp.exp(jnp.where(present, alpha - amax, -30.0))
        exl = jnp.exp(aloop - amax)
        den = jnp.sum(ex, axis=1, keepdims=True) + exl
        xs_h = xs[:, h * C:(h + 1) * C]
        num = lax.dot_general(ex, xs_h, (((1,), (0,)), ((), ())),
                              preferred_element_type=jnp.float32)
        acc = acc + (num + exl * xs_h) / (den + 1e-16)
    return jax.nn.relu(acc * (1.0 / H) + bg)


def _branch(cnt_t, mat_t, x, w1t, b1, wgt, msrc, mdst, msrct, coeff, einv, bg,
            w2t, b2):
    st = cnt_t * mat_t
    xw1 = lax.dot_general(x, w1t, (((1,), (0,)), ((), ())),
                          preferred_element_type=jnp.float32)
    x1 = _gcn(st, xw1, b1)
    xa = _gat(cnt_t, mat_t, x1, wgt, msrc, mdst, msrct, coeff, einv, bg)
    xw2 = lax.dot_general(xa, w2t, (((1,), (0,)), ((), ())),
                          preferred_element_type=jnp.float32)
    x2 = _gcn(st, xw2, b2)
    return x1, x2


def _main_body(e_cc, e_dd,
               cntt_c, matt_c, x_c, w1t_c, b1_c, wgt_c, msrc_c, mdst_c,
               msrct_c, coeff_c, bg_c, w2t_c, b2_c, wct_c, bc_c,
               cntt_d, matt_d, x_d, w1t_d, b1_d, wgt_d, msrc_d, mdst_d,
               msrct_d, coeff_d, bg_d, w2t_d, b2_d, wct_d, bc_d,
               score_ref, cir_ref, dis_ref):
    x1, x2 = _branch(cntt_c[...], matt_c[...], x_c[...], w1t_c[...], b1_c[...],
                     wgt_c[...], msrc_c[...], mdst_c[...], msrct_c[...],
                     coeff_c[...], 1.0 / e_cc, bg_c[...], w2t_c[...], b2_c[...])
    y1, y2 = _branch(cntt_d[...], matt_d[...], x_d[...], w1t_d[...], b1_d[...],
                     wgt_d[...], msrc_d[...], mdst_d[...], msrct_d[...],
                     coeff_d[...], 1.0 / e_dd, bg_d[...], w2t_d[...], b2_d[...])
    fc = jnp.concatenate([x1, x2], axis=1)
    fd = jnp.concatenate([y1, y2], axis=1)
    cir = lax.dot_general(fc, wct_c[...], (((1,), (0,)), ((), ())),
                          preferred_element_type=jnp.float32) + bc_c[...]
    dis = lax.dot_general(fd, wct_d[...], (((1,), (0,)), ((), ())),
                          preferred_element_type=jnp.float32) + bc_d[...]
    cir_ref[...] = cir
    dis_ref[...] = dis
    score_ref[...] = lax.dot_general(cir, dis, (((1,), (1,)), ((), ())),
                                     preferred_element_type=jnp.float32)


def _pad2(a, m0, m1):
    return jnp.pad(a, ((0, m0 - a.shape[0]), (0, m1 - a.shape[1])))


def kernel(cc_matrix, cc_edges, dd_matrix, dd_edges, x_cir, x_dis,
           W1c, b1c, Wgc, asrc_c, adst_c, We_c, ae_c, bg_c, W2c, b2c,
           W1d, b1d, Wgd, asrc_d, adst_d, We_d, ae_d, bg_d, W2d, b2d,
           Wcnn_c, bcnn_c, Wcnn_d, bcnn_d):
    e_cc = cc_edges.shape[1]
    e_dd = dd_edges.shape[1]
    chunk_c = 4736
    epad_c = ((e_cc + chunk_c - 1) // chunk_c) * chunk_c
    ecc = jnp.pad(cc_edges, ((0, 0), (0, epad_c - e_cc)), constant_values=1 << 20)
    chunk_d = ((e_dd + 127) // 128) * 128
    edd = jnp.pad(dd_edges, ((0, 0), (0, chunk_d - e_dd)), constant_values=1 << 20)

    cntt_c = _hist(ecc, M_CIR, chunk_c)
    cntt_d = _hist(edd, M_DIS, chunk_d)

    def prep(matrix, x, W1, Wg, asrc, adst, We, ae, W2, Wcnn, m):
        matt = _pad2(matrix.T, m, m)
        xp = _pad2(x, m, C)
        wg2 = Wg.reshape(H, C, C)
        msrc = jnp.einsum('hck,hc->kh', wg2, asrc)
        mdst = jnp.einsum('hck,hc->kh', wg2, adst)
        coeff = (We.reshape(H, C) * ae).sum(-1)[None, :]
        return (matt, xp, W1.T, Wg.T, msrc, mdst, msrc.T, coeff,
                W2.T, Wcnn.T)

    (matt_c, xp_c, w1t_c, wgt_c, msrc_c, mdst_c, msrct_c, coeff_c,
     w2t_c, wct_c) = prep(cc_matrix, x_cir, W1c, Wgc, asrc_c, adst_c,
                          We_c, ae_c, W2c, Wcnn_c, M_CIR)
    (matt_d, xp_d, w1t_d, wgt_d, msrc_d, mdst_d, msrct_d, coeff_d,
     w2t_d, wct_d) = prep(dd_matrix, x_dis, W1d, Wgd, asrc_d, adst_d,
                          We_d, ae_d, W2d, Wcnn_d, M_DIS)

    out_shapes = (
        jax.ShapeDtypeStruct((M_CIR, M_DIS), jnp.float32),
        jax.ShapeDtypeStruct((M_CIR, 2 * C), jnp.float32),
        jax.ShapeDtypeStruct((M_DIS, 2 * C), jnp.float32),
    )
    score, cir, dis = pl.pallas_call(
        functools.partial(_main_body, float(e_cc), float(e_dd)),
        out_shape=out_shapes,
    )(cntt_c, matt_c, xp_c, w1t_c, b1c[None, :], wgt_c, msrc_c, mdst_c,
      msrct_c, coeff_c, bg_c[None, :], w2t_c, b2c[None, :], wct_c,
      bcnn_c[None, :],
      cntt_d, matt_d, xp_d, w1t_d, b1d[None, :], wgt_d, msrc_d, mdst_d,
      msrct_d, coeff_d, bg_d[None, :], w2t_d, b2d[None, :], wct_d,
      bcnn_d[None, :])
    return (score[:N_CIR, :N_DIS], cir[:N_CIR], dis[:N_DIS])


# speedup vs baseline: 88.6122x; 1.4711x over previous
import functools
import jax
import jax.numpy as jnp
from jax import lax
from jax.experimental import pallas as pl
from jax.experimental.pallas import tpu as pltpu

N_CIR = 585
N_DIS = 88
H = 8
C = 128


def _dot_nt(a, b):
    return lax.dot_general(a, b, (((1,), (1,)), ((), ())),
                           preferred_element_type=jnp.float32)


def _dot_nn(a, b):
    return lax.dot_general(a, b, (((1,), (0,)), ((), ())),
                           preferred_element_type=jnp.float32)


def _hist_t(rows_ref, cols_ref, cnt_ref, n):
    nb, chunk = rows_ref.shape
    for i in range(nb):
        rows = rows_ref[i:i + 1, :]
        cols = cols_ref[i:i + 1, :]
        iota = lax.broadcasted_iota(jnp.int32, (n, chunk), 0)
        rt = jnp.where(iota == jnp.broadcast_to(rows, (n, chunk)),
                       1.0, 0.0).astype(jnp.bfloat16)
        ct = jnp.where(iota == jnp.broadcast_to(cols, (n, chunk)),
                       1.0, 0.0).astype(jnp.bfloat16)
        acc = _dot_nt(ct, rt)
        if i == 0:
            cnt_ref[...] = acc
        else:
            cnt_ref[...] += acc


def _gcn(st, xw, dinv, b):
    out = _dot_nn(st, xw * dinv) * dinv
    return jax.nn.relu(out + dinv * dinv * xw + b)


def _gat(cnt_t, mat_t, st, x1, wg, asrc, adst, we2, ae, einv, bg):
    m = x1.shape[0]
    xs = _dot_nt(x1, wg)
    mean_ea = jnp.sum(st, axis=1, keepdims=True).sum(axis=0, keepdims=True) * einv
    present = cnt_t > 0.0
    acc = jnp.zeros((m, C), jnp.float32)
    for h in range(H):
        xs_h = xs[:, h * C:(h + 1) * C]
        asrc_h = asrc[h:h + 1, :]
        adst_h = adst[h:h + 1, :]
        coeff_h = jnp.sum(we2[h:h + 1, :] * ae[h:h + 1, :], axis=1,
                          keepdims=True)
        a_src_col = jnp.sum(xs_h * asrc_h, axis=1, keepdims=True)
        a_dst_col = jnp.sum(xs_h * adst_h, axis=1, keepdims=True)
        a_src_row = _dot_nt(asrc_h, xs_h)
        alpha = a_dst_col + a_src_row + mat_t * coeff_h
        alpha = jnp.where(alpha > 0, alpha, 0.2 * alpha)
        aloop = a_src_col + a_dst_col + mean_ea * coeff_h
        aloop = jnp.where(aloop > 0, aloop, 0.2 * aloop)
        amax = jnp.max(jnp.where(present, alpha, -1e30), axis=1, keepdims=True)
        amax = jnp.maximum(amax, aloop)
        ex = cnt_t * jnp.exp(jnp.where(present, alpha - amax, -30.0))
        exl = jnp.exp(aloop - amax)
        den = jnp.sum(ex, axis=1, keepdims=True) + exl
        num = _dot_nn(ex, xs_h) + exl * xs_h
        acc = acc + num / (den + 1e-16)
    return jax.nn.relu(acc * (1.0 / H) + bg)


def _branch(cnt_t, mat, x, w1, b1, wg, asrc, adst, we2, ae, einv, bg, w2, b2):
    mat_t = mat.T
    st = cnt_t * mat_t
    dinv = lax.rsqrt(jnp.sum(st, axis=1, keepdims=True) + 1.0)
    x1 = _gcn(st, _dot_nt(x, w1), dinv, b1)
    xa = _gat(cnt_t, mat_t, st, x1, wg, asrc, adst, we2, ae, einv, bg)
    x2 = _gcn(st, _dot_nt(xa, w2), dinv, b2)
    return x1, x2


def _body(e_cc, e_dd,
          rows_c, cols_c, rows_d, cols_d, mat_c, mat_d, x_c, x_d,
          w1c, b1c, wgc, asrc_c, adst_c, we2c, aec, bgc, w2c, b2c,
          w1d, b1d, wgd, asrc_d, adst_d, we2d, aed, bgd, w2d, b2d,
          wcc, bcc, wcd, bcd,
          score_ref, cir_ref, dis_ref, cntc_ref, cntd_ref):
    _hist_t(rows_c, cols_c, cntc_ref, N_CIR)
    _hist_t(rows_d, cols_d, cntd_ref, N_DIS)
    x1, x2 = _branch(cntc_ref[...], mat_c[...], x_c[...], w1c[...], b1c[...],
                     wgc[...], asrc_c[...], adst_c[...], we2c[...], aec[...],
                     1.0 / e_cc, bgc[...], w2c[...], b2c[...])
    y1, y2 = _branch(cntd_ref[...], mat_d[...], x_d[...], w1d[...], b1d[...],
                     wgd[...], asrc_d[...], adst_d[...], we2d[...], aed[...],
                     1.0 / e_dd, bgd[...], w2d[...], b2d[...])
    cir = _dot_nt(jnp.concatenate([x1, x2], axis=1), wcc[...]) + bcc[...]
    dis = _dot_nt(jnp.concatenate([y1, y2], axis=1), wcd[...]) + bcd[...]
    cir_ref[...] = cir
    dis_ref[...] = dis
    score_ref[...] = _dot_nt(cir, dis)


def kernel(cc_matrix, cc_edges, dd_matrix, dd_edges, x_cir, x_dis,
           W1c, b1c, Wgc, asrc_c, adst_c, We_c, ae_c, bg_c, W2c, b2c,
           W1d, b1d, Wgd, asrc_d, adst_d, We_d, ae_d, bg_d, W2d, b2d,
           Wcnn_c, bcnn_c, Wcnn_d, bcnn_d):
    e_cc = cc_edges.shape[1]
    e_dd = dd_edges.shape[1]
    nb_c = 8
    nb_d = 2
    rows_c = cc_edges[0].reshape(nb_c, e_cc // nb_c)
    cols_c = cc_edges[1].reshape(nb_c, e_cc // nb_c)
    rows_d = dd_edges[0].reshape(nb_d, e_dd // nb_d)
    cols_d = dd_edges[1].reshape(nb_d, e_dd // nb_d)

    out_shapes = (
        jax.ShapeDtypeStruct((N_CIR, N_DIS), jnp.float32),
        jax.ShapeDtypeStruct((N_CIR, 2 * C), jnp.float32),
        jax.ShapeDtypeStruct((N_DIS, 2 * C), jnp.float32),
    )
    return pl.pallas_call(
        functools.partial(_body, float(e_cc), float(e_dd)),
        out_shape=out_shapes,
        scratch_shapes=[pltpu.VMEM((N_CIR, N_CIR), jnp.float32),
                        pltpu.VMEM((N_DIS, N_DIS), jnp.float32)],
    )(rows_c, cols_c, rows_d, cols_d, cc_matrix, dd_matrix, x_cir, x_dis,
      W1c, b1c[None, :], Wgc, asrc_c, adst_c, We_c.reshape(H, C), ae_c,
      bg_c[None, :], W2c, b2c[None, :],
      W1d, b1d[None, :], Wgd, asrc_d, adst_d, We_d.reshape(H, C), ae_d,
      bg_d[None, :], W2d, b2d[None, :],
      Wcnn_c, bcnn_c[None, :], Wcnn_d, bcnn_d[None, :])


# speedup vs baseline: 97.6181x; 1.1016x over previous
import functools
import jax
import jax.numpy as jnp
from jax import lax
from jax.experimental import pallas as pl
from jax.experimental.pallas import tpu as pltpu

N_CIR = 585
N_DIS = 88
H = 8
C = 128


def _dot_nt(a, b):
    return lax.dot_general(a, b, (((1,), (1,)), ((), ())),
                           preferred_element_type=jnp.float32)


def _dot_nn(a, b):
    return lax.dot_general(a, b, (((1,), (0,)), ((), ())),
                           preferred_element_type=jnp.float32)


def _hist_t(edges_ref, cnt_ref, n, nb):
    e = edges_ref.shape[1]
    chunk = e // nb
    r_all = edges_ref[0:1, :]
    c_all = edges_ref[1:2, :]
    for i in range(nb):
        rows = r_all[:, i * chunk:(i + 1) * chunk]
        cols = c_all[:, i * chunk:(i + 1) * chunk]
        iota = lax.broadcasted_iota(jnp.int32, (n, chunk), 0)
        rt = jnp.where(iota == jnp.broadcast_to(rows, (n, chunk)),
                       1.0, 0.0).astype(jnp.bfloat16)
        ct = jnp.where(iota == jnp.broadcast_to(cols, (n, chunk)),
                       1.0, 0.0).astype(jnp.bfloat16)
        acc = _dot_nt(ct, rt)
        if i == 0:
            cnt_ref[...] = acc
        else:
            cnt_ref[...] += acc


def _gcn(st, xw, dinv, b):
    out = _dot_nn(st, xw * dinv) * dinv
    return jax.nn.relu(out + dinv * dinv * xw + b)


def _gat(cnt_t, mat_t, st, x1, wg, asrc, adst, we2, ae, einv, bg):
    m = x1.shape[0]
    xs = _dot_nt(x1, wg)
    mean_ea = jnp.sum(st, axis=1, keepdims=True).sum(axis=0, keepdims=True) * einv
    present = cnt_t > 0.0
    acc = jnp.zeros((m, C), jnp.float32)
    for h in range(H):
        xs_h = xs[:, h * C:(h + 1) * C]
        asrc_h = asrc[h:h + 1, :]
        adst_h = adst[h:h + 1, :]
        coeff_h = jnp.sum(we2[h:h + 1, :] * ae[h:h + 1, :], axis=1,
                          keepdims=True)
        a_src_col = jnp.sum(xs_h * asrc_h, axis=1, keepdims=True)
        a_dst_col = jnp.sum(xs_h * adst_h, axis=1, keepdims=True)
        a_src_row = _dot_nt(asrc_h, xs_h)
        alpha = a_dst_col + a_src_row + mat_t * coeff_h
        alpha = jnp.where(alpha > 0, alpha, 0.2 * alpha)
        aloop = a_src_col + a_dst_col + mean_ea * coeff_h
        aloop = jnp.where(aloop > 0, aloop, 0.2 * aloop)
        amax = jnp.max(jnp.where(present, alpha, -1e30), axis=1, keepdims=True)
        amax = jnp.maximum(amax, aloop)
        ex = cnt_t * jnp.exp(jnp.where(present, alpha - amax, -30.0))
        exl = jnp.exp(aloop - amax)
        den = jnp.sum(ex, axis=1, keepdims=True) + exl
        num = _dot_nn(ex, xs_h) + exl * xs_h
        acc = acc + num / (den + 1e-16)
    return jax.nn.relu(acc * (1.0 / H) + bg)


def _branch(cnt_t, mat, x, w1, b1, wg, asrc, adst, we2, ae, einv, bg, w2, b2):
    mat_t = mat.T
    st = cnt_t * mat_t
    dinv = lax.rsqrt(jnp.sum(st, axis=1, keepdims=True) + 1.0)
    x1 = _gcn(st, _dot_nt(x, w1), dinv, b1)
    xa = _gat(cnt_t, mat_t, st, x1, wg, asrc, adst, we2, ae, einv, bg)
    x2 = _gcn(st, _dot_nt(xa, w2), dinv, b2)
    return x1, x2


def _body(e_cc, e_dd,
          edges_c, edges_d, mat_c, mat_d, x_c, x_d,
          w1c, b1c, wgc, asrc_c, adst_c, we2c, aec, bgc, w2c, b2c,
          w1d, b1d, wgd, asrc_d, adst_d, we2d, aed, bgd, w2d, b2d,
          wcc, bcc, wcd, bcd,
          score_ref, cir_ref, dis_ref, cntc_ref, cntd_ref):
    _hist_t(edges_c, cntc_ref, N_CIR, 8)
    _hist_t(edges_d, cntd_ref, N_DIS, 2)
    x1, x2 = _branch(cntc_ref[...], mat_c[...], x_c[...], w1c[...],
                     b1c[...][None, :],
                     wgc[...], asrc_c[...], adst_c[...], we2c[...], aec[...],
                     1.0 / e_cc, bgc[...][None, :], w2c[...],
                     b2c[...][None, :])
    y1, y2 = _branch(cntd_ref[...], mat_d[...], x_d[...], w1d[...],
                     b1d[...][None, :],
                     wgd[...], asrc_d[...], adst_d[...], we2d[...], aed[...],
                     1.0 / e_dd, bgd[...][None, :], w2d[...],
                     b2d[...][None, :])
    cir = _dot_nt(jnp.concatenate([x1, x2], axis=1), wcc[...]) + bcc[...][None, :]
    dis = _dot_nt(jnp.concatenate([y1, y2], axis=1), wcd[...]) + bcd[...][None, :]
    cir_ref[...] = cir
    dis_ref[...] = dis
    score_ref[...] = _dot_nt(cir, dis)


def kernel(cc_matrix, cc_edges, dd_matrix, dd_edges, x_cir, x_dis,
           W1c, b1c, Wgc, asrc_c, adst_c, We_c, ae_c, bg_c, W2c, b2c,
           W1d, b1d, Wgd, asrc_d, adst_d, We_d, ae_d, bg_d, W2d, b2d,
           Wcnn_c, bcnn_c, Wcnn_d, bcnn_d):
    e_cc = cc_edges.shape[1]
    e_dd = dd_edges.shape[1]

    out_shapes = (
        jax.ShapeDtypeStruct((N_CIR, N_DIS), jnp.float32),
        jax.ShapeDtypeStruct((N_CIR, 2 * C), jnp.float32),
        jax.ShapeDtypeStruct((N_DIS, 2 * C), jnp.float32),
    )
    return pl.pallas_call(
        functools.partial(_body, float(e_cc), float(e_dd)),
        out_shape=out_shapes,
        scratch_shapes=[pltpu.VMEM((N_CIR, N_CIR), jnp.float32),
                        pltpu.VMEM((N_DIS, N_DIS), jnp.float32)],
    )(cc_edges, dd_edges, cc_matrix, dd_matrix, x_cir, x_dis,
      W1c, b1c, Wgc, asrc_c, adst_c, We_c.reshape(H, C), ae_c,
      bg_c, W2c, b2c,
      W1d, b1d, Wgd, asrc_d, adst_d, We_d.reshape(H, C), ae_d,
      bg_d, W2d, b2d,
      Wcnn_c, bcnn_c, Wcnn_d, bcnn_d)
